# baseline (speedup 1.0000x reference)
import functools

import jax
import jax.numpy as jnp
from jax import lax
from jax.experimental import pallas as pl
from jax.experimental.pallas import tpu as pltpu
from jax.experimental.pallas import tpu_sc as plsc

SEQ_LEN = 2048
HIDDEN = 64
NUM_CORES = 2
NUM_SUBCORES = 16
NUM_WORKERS = NUM_CORES * NUM_SUBCORES
CHUNK = 256


@functools.partial(jax.jit, static_argnums=(2,))
def _sc_gather(idx2d, table, total):
  per_w = total // NUM_WORKERS
  n_chunks = per_w // CHUNK
  assert per_w % CHUNK == 0 and n_chunks % 4 == 0 and n_chunks >= 8
  mesh = plsc.VectorSubcoreMesh(core_axis_name="c", subcore_axis_name="s")

  @functools.partial(
      pl.kernel,
      out_type=jax.ShapeDtypeStruct((total, HIDDEN), jnp.float32),
      mesh=mesh,
      scratch_types=[
          pltpu.VMEM((n_chunks, CHUNK), jnp.int32),
          pltpu.VMEM((4, CHUNK, HIDDEN), jnp.float32),
          [pltpu.SemaphoreType.DMA] * 4,
          [pltpu.SemaphoreType.DMA] * 4,
      ],
      compiler_params=pltpu.CompilerParams(use_tc_tiling_on_sc=False),
  )
  def k(idx_hbm, table_hbm, out_hbm, idx_v, rows_v, gsem, osem):
    wid = lax.axis_index("s") * NUM_CORES + lax.axis_index("c")
    base = wid * per_w
    pltpu.sync_copy(idx_hbm.at[pl.ds(wid * n_chunks, n_chunks)], idx_v)

    def gather_start(j, s):
      pltpu.async_copy(table_hbm.at[idx_v.at[j]], rows_v.at[s], gsem[s])

    def gather_wait(j, s):
      pltpu.make_async_copy(
          table_hbm.at[idx_v.at[j]], rows_v.at[s], gsem[s]
      ).wait()

    def store_start(j, s):
      pltpu.async_copy(
          rows_v.at[s], out_hbm.at[pl.ds(base + j * CHUNK, CHUNK)], osem[s]
      )

    def store_wait(j, s):
      pltpu.make_async_copy(
          rows_v.at[s], out_hbm.at[pl.ds(base + j * CHUNK, CHUNK)], osem[s]
      ).wait()

    gather_start(0, 0)
    gather_start(1, 1)
    for s in range(4):
      if s >= 2:
        store_wait(s - 2, s - 2)
        gather_start(s + 2, (s + 2) % 4)
      else:
        gather_start(s + 2, s + 2)
      gather_wait(s, s)
      store_start(s, s)

    def group(p, _):
      j0 = p * 4
      for s in range(4):
        j = j0 + s
        t = (s + 2) % 4
        store_wait(j - 2, t)
        gather_start(j + 2, t)
        gather_wait(j, s)
        store_start(j, s)
      return 0

    lax.fori_loop(1, n_chunks // 4 - 1, group, 0)

    j0 = n_chunks - 4
    for s in range(4):
      j = j0 + s
      t = (s + 2) % 4
      store_wait(j - 2, t)
      if s < 2:
        gather_start(j + 2, t)
      gather_wait(j, s)
      store_start(j, s)
    store_wait(n_chunks - 2, 2)
    store_wait(n_chunks - 1, 3)

  return k(idx2d, table)


def kernel(time_idx, positions):
  total = time_idx.shape[0] * time_idx.shape[1]
  idx2d = time_idx.reshape(total // CHUNK, CHUNK)
  out = _sc_gather(idx2d, positions, total)
  return out.reshape(time_idx.shape[0], time_idx.shape[1], HIDDEN)


# speedup vs baseline: 1.0028x; 1.0028x over previous
import functools

import jax
import jax.numpy as jnp
from jax import lax
from jax.experimental import pallas as pl
from jax.experimental.pallas import tpu as pltpu
from jax.experimental.pallas import tpu_sc as plsc

HIDDEN = 64
NUM_CORES = 2
NUM_SUBCORES = 16
NUM_WORKERS = NUM_CORES * NUM_SUBCORES


@jax.jit
def _sc_gather(time_idx, table):
  batch, lookup = time_idx.shape
  rows_per_w = batch // NUM_WORKERS
  assert batch % NUM_WORKERS == 0 and rows_per_w % 4 == 0 and rows_per_w >= 8
  mesh = plsc.VectorSubcoreMesh(core_axis_name="c", subcore_axis_name="s")

  @functools.partial(
      pl.kernel,
      out_type=jax.ShapeDtypeStruct((batch, lookup, HIDDEN), jnp.float32),
      mesh=mesh,
      scratch_types=[
          pltpu.VMEM((rows_per_w, lookup), jnp.int32),
          pltpu.VMEM((4, lookup, HIDDEN), jnp.float32),
          [pltpu.SemaphoreType.DMA] * 4,
          [pltpu.SemaphoreType.DMA] * 4,
      ],
      compiler_params=pltpu.CompilerParams(use_tc_tiling_on_sc=False),
  )
  def k(idx_hbm, table_hbm, out_hbm, idx_v, rows_v, gsem, osem):
    wid = lax.axis_index("s") * NUM_CORES + lax.axis_index("c")
    base = wid * rows_per_w
    pltpu.sync_copy(idx_hbm.at[pl.ds(base, rows_per_w)], idx_v)

    def gather_start(j, s):
      pltpu.async_copy(table_hbm.at[idx_v.at[j]], rows_v.at[s], gsem[s])

    def gather_wait(j, s):
      pltpu.make_async_copy(
          table_hbm.at[idx_v.at[j]], rows_v.at[s], gsem[s]
      ).wait()

    def store_start(j, s):
      pltpu.async_copy(rows_v.at[s], out_hbm.at[base + j], osem[s])

    def store_wait(j, s):
      pltpu.make_async_copy(
          rows_v.at[s], out_hbm.at[base + j], osem[s]
      ).wait()

    gather_start(0, 0)
    gather_start(1, 1)
    for s in range(4):
      if s >= 2:
        store_wait(s - 2, s - 2)
        gather_start(s + 2, (s + 2) % 4)
      else:
        gather_start(s + 2, s + 2)
      gather_wait(s, s)
      store_start(s, s)

    def group(p, _):
      j0 = p * 4
      for s in range(4):
        j = j0 + s
        t = (s + 2) % 4
        store_wait(j - 2, t)
        gather_start(j + 2, t)
        gather_wait(j, s)
        store_start(j, s)
      return 0

    lax.fori_loop(1, rows_per_w // 4 - 1, group, 0)

    j0 = rows_per_w - 4
    for s in range(4):
      j = j0 + s
      t = (s + 2) % 4
      store_wait(j - 2, t)
      if s < 2:
        gather_start(j + 2, t)
      gather_wait(j, s)
      store_start(j, s)
    store_wait(rows_per_w - 2, 2)
    store_wait(rows_per_w - 1, 3)

  return k(time_idx, table)


def kernel(time_idx, positions):
  return _sc_gather(time_idx, positions)


# speedup vs baseline: 1.1193x; 1.1162x over previous
import functools

import jax
import jax.numpy as jnp
from jax import lax
from jax.experimental import pallas as pl
from jax.experimental.pallas import tpu as pltpu
from jax.experimental.pallas import tpu_sc as plsc

HIDDEN = 64
NUM_CORES = 2
NUM_SUBCORES = 16
NUM_WORKERS = NUM_CORES * NUM_SUBCORES
IW = 128


@functools.partial(jax.jit, static_argnums=(3, 4))
def _sc_gather(idx_a, idx_b, table_pad, batch, lookup):
  rows_per_w = batch // NUM_WORKERS
  tail = lookup - IW
  assert batch % NUM_WORKERS == 0 and rows_per_w % 2 == 0 and rows_per_w >= 4
  mesh = plsc.VectorSubcoreMesh(core_axis_name="c", subcore_axis_name="s")

  @functools.partial(
      pl.kernel,
      out_type=jax.ShapeDtypeStruct((batch, lookup, HIDDEN), jnp.float32),
      mesh=mesh,
      scratch_types=[
          pltpu.VMEM((rows_per_w, IW), jnp.int32),
          pltpu.VMEM((rows_per_w, tail), jnp.int32),
          pltpu.VMEM((2, lookup, IW), jnp.float32),
          pltpu.VMEM((lookup, HIDDEN), jnp.float32),
          [pltpu.SemaphoreType.DMA] * 2,
          pltpu.SemaphoreType.DMA,
      ],
  )
  def k(ia_hbm, ib_hbm, table_hbm, out_hbm, ia_v, ib_v, raw_v, st_v, gsem,
        osem):
    wid = lax.axis_index("s") * NUM_CORES + lax.axis_index("c")
    base = wid * rows_per_w
    pltpu.sync_copy(ia_hbm.at[pl.ds(base, rows_per_w)], ia_v)
    pltpu.sync_copy(ib_hbm.at[pl.ds(base, rows_per_w)], ib_v)

    def gather_start(j, s):
      pltpu.async_copy(
          table_hbm.at[ia_v.at[j]], raw_v.at[s, pl.ds(0, IW)], gsem[s]
      )
      pltpu.async_copy(
          table_hbm.at[ib_v.at[j]], raw_v.at[s, pl.ds(IW, tail)], gsem[s]
      )

    def gather_wait(j, s):
      pltpu.make_async_copy(
          table_hbm.at[ia_v.at[j]], raw_v.at[s, pl.ds(0, IW)], gsem[s]
      ).wait()
      pltpu.make_async_copy(
          table_hbm.at[ib_v.at[j]], raw_v.at[s, pl.ds(IW, tail)], gsem[s]
      ).wait()

    def vcopy(s):
      def body(r8, _):
        for rr in range(8):
          r = r8 * 8 + rr
          for c in range(HIDDEN // 16):
            st_v[r, pl.ds(c * 16, 16)] = raw_v[s, r, pl.ds(c * 16, 16)]
        return 0

      lax.fori_loop(0, lookup // 8, body, 0)

    def store_start(j):
      pltpu.async_copy(st_v, out_hbm.at[base + j], osem)

    def store_wait(j):
      pltpu.make_async_copy(st_v, out_hbm.at[base + j], osem).wait()

    def step(j, s, first, last):
      if not last:
        gather_start(j + 1, 1 - s)
      if not first:
        store_wait(j - 1)
      gather_wait(j, s)
      vcopy(s)
      store_start(j)

    gather_start(0, 0)
    step(0, 0, True, False)
    step(1, 1, False, False)

    def group(p, _):
      j0 = p * 2
      step(j0, 0, False, False)
      step(j0 + 1, 1, False, False)
      return 0

    lax.fori_loop(1, rows_per_w // 2 - 1, group, 0)

    j0 = rows_per_w - 2
    step(j0, 0, False, False)
    step(j0 + 1, 1, False, True)
    store_wait(rows_per_w - 1)

  return k(idx_a, idx_b, table_pad)


def kernel(time_idx, positions):
  batch, lookup = time_idx.shape
  idx_a = time_idx[:, :IW]
  idx_b = time_idx[:, IW:]
  table_pad = jnp.pad(positions, ((0, 0), (0, IW - HIDDEN)))
  return _sc_gather(idx_a, idx_b, table_pad, batch, lookup)


# speedup vs baseline: 1.3564x; 1.2119x over previous
import functools

import jax
import jax.numpy as jnp
from jax import lax
from jax.experimental import pallas as pl
from jax.experimental.pallas import tpu as pltpu
from jax.experimental.pallas import tpu_sc as plsc

SEQ = 2048
HIDDEN = 64
NUM_CORES = 2
NUM_SUBCORES = 16
NUM_WORKERS = NUM_CORES * NUM_SUBCORES
HG = 8
LGROUPS = NUM_WORKERS // (HIDDEN // HG)


@functools.partial(jax.jit, static_argnums=(2, 3))
def _sc_gather(idx_t, table_t, batch, lookup):
  n_units = lookup // LGROUPS
  assert lookup % LGROUPS == 0 and n_units % 2 == 0 and batch % 16 == 0
  groups = batch // 16
  mesh = plsc.VectorSubcoreMesh(core_axis_name="c", subcore_axis_name="s")

  @functools.partial(
      pl.kernel,
      out_type=jax.ShapeDtypeStruct((lookup, HIDDEN, batch), jnp.float32),
      mesh=mesh,
      scratch_types=[
          pltpu.VMEM((HG, SEQ), jnp.float32),
          pltpu.VMEM((2, batch), jnp.int32),
          pltpu.VMEM((2, HG, batch), jnp.float32),
          [pltpu.SemaphoreType.DMA] * 2,
          [pltpu.SemaphoreType.DMA] * 2,
      ],
      compiler_params=pltpu.CompilerParams(needs_layout_passes=False),
  )
  def k(idx_hbm, table_hbm, out_hbm, table_v, idx_v, blk_v, isem, osem):
    wid = lax.axis_index("s") * NUM_CORES + lax.axis_index("c")
    hg = lax.rem(wid, HIDDEN // HG)
    lphase = wid // (HIDDEN // HG)
    pltpu.sync_copy(table_hbm.at[pl.ds(hg * HG, HG)], table_v)

    def lpos(k_):
      return lphase + k_ * LGROUPS

    def idx_start(k_, s):
      pltpu.async_copy(idx_hbm.at[lpos(k_)], idx_v.at[s], isem[s])

    def idx_wait(k_, s):
      pltpu.make_async_copy(idx_hbm.at[lpos(k_)], idx_v.at[s], isem[s]).wait()

    def store_start(k_, s):
      pltpu.async_copy(
          blk_v.at[s], out_hbm.at[lpos(k_), pl.ds(hg * HG, HG)], osem[s]
      )

    def store_wait(k_, s):
      pltpu.make_async_copy(
          blk_v.at[s], out_hbm.at[lpos(k_), pl.ds(hg * HG, HG)], osem[s]
      ).wait()

    h_vecs = [jnp.full((16,), h, jnp.int32) for h in range(HG)]

    def assemble(s):
      def body(g, _):
        seq = idx_v[s, pl.ds(g * 16, 16)]
        for h in range(HG):
          blk_v[s, h, pl.ds(g * 16, 16)] = plsc.load_gather(
              table_v, [h_vecs[h], seq]
          )
        return 0

      lax.fori_loop(0, groups, body, 0, unroll=2)

    def step(k_, s, first, last):
      if not last:
        idx_start(k_ + 1, 1 - s)
      idx_wait(k_, s)
      if not first:
        store_wait(k_ - 2 if k_ >= 2 else k_, s)
      assemble(s)
      store_start(k_, s)

    idx_start(0, 0)
    idx_start(1, 1)
    idx_wait(0, 0)
    assemble(0)
    store_start(0, 0)
    idx_start(2, 0)
    idx_wait(1, 1)
    assemble(1)
    store_start(1, 1)

    def group_body(p, _):
      k0 = p * 2
      idx_start(k0 + 1, 1)
      idx_wait(k0, 0)
      store_wait(k0 - 2, 0)
      assemble(0)
      store_start(k0, 0)
      idx_start(k0 + 2, 0)
      idx_wait(k0 + 1, 1)
      store_wait(k0 - 1, 1)
      assemble(1)
      store_start(k0 + 1, 1)
      return 0

    lax.fori_loop(1, n_units // 2 - 1, group_body, 0)

    k0 = n_units - 2
    idx_start(k0 + 1, 1)
    idx_wait(k0, 0)
    store_wait(k0 - 2, 0)
    assemble(0)
    store_start(k0, 0)
    idx_wait(k0 + 1, 1)
    store_wait(k0 - 1, 1)
    assemble(1)
    store_start(k0 + 1, 1)
    store_wait(k0, 0)
    store_wait(k0 + 1, 1)

  return k(idx_t, table_t)


def kernel(time_idx, positions):
  batch, lookup = time_idx.shape
  idx_t = time_idx.T
  table_t = positions.T
  out_t = _sc_gather(idx_t, table_t, batch, lookup)
  return jnp.transpose(out_t, (2, 0, 1))
